# Initial kernel scaffold; baseline (speedup 1.0000x reference)
import functools

import jax
import jax.numpy as jnp
from jax import lax
from jax.experimental import pallas as pl
from jax.experimental.pallas import tpu as pltpu
from jax.experimental.pallas import tpu_sc as plsc

BN = 2048
CHUNK = 1024
IDXW = 128
NSUB = 32


def _node_body(nf_ref, pos_ref, ids_ref, w1_ref, wp_ref, w2_ref, b1_ref, b2_ref,
               eacc_ref, f_ref):
    i = pl.program_id(0)
    nf = nf_ref[...]
    p = pos_ref[...]
    ids = ids_ref[0]
    w1 = w1_ref[...]
    wp = wp_ref[...]
    w2 = w2_ref[...]
    b1 = b1_ref[...]
    b2 = b2_ref[0, 0]

    wpw1 = jax.lax.dot_general(wp, w1, (((1,), (0,)), ((), ())),
                               preferred_element_type=jnp.float32)
    z = (jax.lax.dot_general(nf, w1, (((1,), (0,)), ((), ())),
                             preferred_element_type=jnp.float32)
         + jax.lax.dot_general(p, wpw1, (((1,), (0,)), ((), ())),
                               preferred_element_type=jnp.float32)
         + b1)
    sg = 1.0 / (1.0 + jnp.exp(-z))
    h = z * sg
    node_e = jnp.sum(h * w2, axis=1, keepdims=True) + b2

    valid = ids >= 0
    node_e = jnp.where(valid, node_e, 0.0)
    vals = jnp.concatenate([node_e, valid.astype(jnp.float32)], axis=1)

    g = eacc_ref.shape[0]
    onehot = (ids == jax.lax.broadcasted_iota(jnp.int32, (nf.shape[0], g), 1)
              ).astype(jnp.float32)
    contrib = jax.lax.dot_general(onehot, vals, (((0,), (0,)), ((), ())),
                                  preferred_element_type=jnp.float32)

    @pl.when(i == 0)
    def _():
        eacc_ref[...] = jnp.zeros_like(eacc_ref)

    eacc_ref[...] += contrib

    dsilu = sg * (1.0 + z * (1.0 - sg))
    g2 = dsilu * w2
    grad_p = jax.lax.dot_general(g2, wpw1, (((1,), (1,)), ((), ())),
                                 preferred_element_type=jnp.float32)
    f_ref[...] = jnp.concatenate(
        [grad_p, jnp.zeros((grad_p.shape[0], 1), jnp.float32)], axis=1)


def _node_call(nf, pos, ids3, w1, wp, w2r, b1r, b2r, g):
    n, d = nf.shape
    nb = ids3.shape[0]
    npad = nb * BN
    h = w1.shape[1]
    return pl.pallas_call(
        _node_body,
        grid=(nb,),
        in_specs=[
            pl.BlockSpec((BN, d), lambda i: (i, 0)),
            pl.BlockSpec((BN, 3), lambda i: (i, 0)),
            pl.BlockSpec((1, BN, 1), lambda i: (i, 0, 0)),
            pl.BlockSpec((d, h), lambda i: (0, 0)),
            pl.BlockSpec((3, d), lambda i: (0, 0)),
            pl.BlockSpec((1, h), lambda i: (0, 0)),
            pl.BlockSpec((1, h), lambda i: (0, 0)),
            pl.BlockSpec((1, 1), lambda i: (0, 0)),
        ],
        out_specs=[
            pl.BlockSpec((g, 2), lambda i: (0, 0)),
            pl.BlockSpec((BN, 4), lambda i: (i, 0)),
        ],
        out_shape=[
            jax.ShapeDtypeStruct((g, 2), jnp.float32),
            jax.ShapeDtypeStruct((npad, 4), jnp.float32),
        ],
    )(nf, pos, ids3, w1, wp, w2r, b1r, b2r)


def _make_edge_kernel(e, n, g):
    nchunks = e // CHUNK
    chunks_per = -(-nchunks // NSUB)
    g9 = g * 9
    mesh = plsc.VectorSubcoreMesh(core_axis_name="c", subcore_axis_name="s")

    @functools.partial(
        pl.kernel,
        mesh=mesh,
        out_type=jax.ShapeDtypeStruct((NSUB, g9), jnp.float32),
        scratch_types=[
            pltpu.VMEM((n,), jnp.int32),
            pltpu.VMEM((CHUNK,), jnp.int32),
            pltpu.VMEM((CHUNK // IDXW, IDXW), jnp.int32),
            pltpu.VMEM((CHUNK, 3), jnp.float32),
            pltpu.VMEM((CHUNK, 4), jnp.float32),
            pltpu.VMEM((g9,), jnp.float32),
            pltpu.SemaphoreType.DMA,
        ],
    )
    def edge_kernel(ei_hbm, ea_hbm, f_hbm, batch_hbm, zero_hbm, out_hbm,
                    batch_v, row_v, col_v, a_v, f_v, acc_v, sem):
        cid0 = lax.axis_index("s") * 2 + lax.axis_index("c")
        pltpu.sync_copy(batch_hbm, batch_v)
        pltpu.sync_copy(zero_hbm, acc_v)
        iota = lax.iota(jnp.int32, 16)
        ndma = CHUNK // IDXW

        def chunk_body(ci, carry):
            cid = ci * NSUB + cid0

            @pl.when(cid < nchunks)
            def _():
                start = cid * CHUNK
                pltpu.sync_copy(ei_hbm.at[0, pl.ds(start, CHUNK)], row_v)
                for j in range(ndma):
                    pltpu.sync_copy(ei_hbm.at[1, pl.ds(start + j * IDXW, IDXW)],
                                    col_v.at[j])
                pltpu.sync_copy(ea_hbm.at[pl.ds(start, CHUNK)], a_v)
                copies = [
                    pltpu.async_copy(f_hbm.at[col_v.at[j]],
                                     f_v.at[pl.ds(j * IDXW, IDXW)], sem)
                    for j in range(ndma)
                ]
                for c in copies:
                    c.wait()

                def vec_body(v, carry2):
                    base = v * 16
                    lane = iota + base
                    r16 = row_v[pl.ds(base, 16)]
                    gid = plsc.load_gather(batch_v, [r16])
                    gid9 = gid * 9
                    a = [plsc.load_gather(a_v, [lane, jnp.full((16,), k, jnp.int32)])
                         for k in range(3)]
                    f = [plsc.load_gather(f_v, [lane, jnp.full((16,), k, jnp.int32)])
                         for k in range(3)]
                    for ii in range(3):
                        for jj in range(3):
                            plsc.addupdate_scatter(
                                acc_v, [gid9 + (3 * ii + jj)], a[ii] * f[jj])
                    return carry2

                lax.fori_loop(0, CHUNK // 16, vec_body, 0)
            return carry

        lax.fori_loop(0, chunks_per, chunk_body, 0)
        pltpu.sync_copy(acc_v, out_hbm.at[cid0])

    return edge_kernel


def _combine_body(p_ref, c_ref, out_ref):
    s = jnp.sum(p_ref[...], axis=0)
    c = c_ref[...]
    det = (c[:, 0] * (c[:, 4] * c[:, 8] - c[:, 5] * c[:, 7])
           - c[:, 1] * (c[:, 3] * c[:, 8] - c[:, 5] * c[:, 6])
           + c[:, 2] * (c[:, 3] * c[:, 7] - c[:, 4] * c[:, 6]))
    vol = jnp.maximum(jnp.abs(det), 1e-10)
    out_ref[...] = s / vol[:, None]


def _combine_call(partials, cell9, g):
    return pl.pallas_call(
        _combine_body,
        grid=(1,),
        in_specs=[
            pl.BlockSpec((NSUB, g, 9), lambda i: (0, 0, 0)),
            pl.BlockSpec((g, 9), lambda i: (0, 0)),
        ],
        out_specs=pl.BlockSpec((g, 9), lambda i: (0, 0)),
        out_shape=jax.ShapeDtypeStruct((g, 9), jnp.float32),
    )(partials, cell9)


def kernel(node_feats, pos, edge_index, edge_attr, batch, cell, W1, b1, W2, b2, Wp):
    n, d = node_feats.shape
    e = edge_index.shape[1]
    g = cell.shape[0]
    h = W1.shape[1]

    nb = -(-n // BN)
    npad = nb * BN
    ids3 = jnp.pad(batch, (0, npad - n), constant_values=-1).reshape(nb, BN, 1)
    w2r = W2.reshape(1, h)
    b1r = b1.reshape(1, h)
    b2r = b2.reshape(1, 1)

    eacc, fpad = _node_call(node_feats, pos, ids3, W1, Wp, w2r, b1r, b2r, g)

    zero = jnp.zeros((g * 9,), jnp.float32)
    edge_kernel = _make_edge_kernel(e, n, g)
    partials = edge_kernel(edge_index, edge_attr, fpad, batch, zero)

    stress9 = _combine_call(partials.reshape(NSUB, g, 9), cell.reshape(g, 9), g)
    return eacc[:, 0], eacc[:, 1], stress9.reshape(g, 3, 3)


# speedup vs baseline: 4.3408x; 4.3408x over previous
import functools

import jax
import jax.numpy as jnp
from jax import lax
from jax.experimental import pallas as pl
from jax.experimental.pallas import tpu as pltpu
from jax.experimental.pallas import tpu_sc as plsc

BN = 2048
CHUNK = 1024
IDXW = 128
NSUB = 32


def _node_body(nf_ref, pos_ref, ids_ref, w1_ref, wp_ref, w2_ref, b1_ref, b2_ref,
               eacc_ref, f_ref):
    i = pl.program_id(0)
    nf = nf_ref[...]
    p = pos_ref[...]
    ids = ids_ref[0]
    w1 = w1_ref[...]
    wp = wp_ref[...]
    w2 = w2_ref[...]
    b1 = b1_ref[...]
    b2 = b2_ref[0, 0]

    wpw1 = jax.lax.dot_general(wp, w1, (((1,), (0,)), ((), ())),
                               preferred_element_type=jnp.float32)
    z = (jax.lax.dot_general(nf, w1, (((1,), (0,)), ((), ())),
                             preferred_element_type=jnp.float32)
         + jax.lax.dot_general(p, wpw1, (((1,), (0,)), ((), ())),
                               preferred_element_type=jnp.float32)
         + b1)
    sg = 1.0 / (1.0 + jnp.exp(-z))
    h = z * sg
    node_e = jnp.sum(h * w2, axis=1, keepdims=True) + b2

    valid = ids >= 0
    node_e = jnp.where(valid, node_e, 0.0)
    vals = jnp.concatenate([node_e, valid.astype(jnp.float32)], axis=1)

    g = eacc_ref.shape[0]
    onehot = (ids == jax.lax.broadcasted_iota(jnp.int32, (nf.shape[0], g), 1)
              ).astype(jnp.float32)
    contrib = jax.lax.dot_general(onehot, vals, (((0,), (0,)), ((), ())),
                                  preferred_element_type=jnp.float32)

    @pl.when(i == 0)
    def _():
        eacc_ref[...] = jnp.zeros_like(eacc_ref)

    eacc_ref[...] += contrib

    dsilu = sg * (1.0 + z * (1.0 - sg))
    g2 = dsilu * w2
    f_ref[...] = jax.lax.dot_general(wpw1, g2, (((1,), (1,)), ((), ())),
                                     preferred_element_type=jnp.float32)


def _node_call(nf, pos, ids3, w1, wp, w2r, b1r, b2r, g):
    n, d = nf.shape
    nb = ids3.shape[0]
    npad = nb * BN
    h = w1.shape[1]
    return pl.pallas_call(
        _node_body,
        grid=(nb,),
        in_specs=[
            pl.BlockSpec((BN, d), lambda i: (i, 0)),
            pl.BlockSpec((BN, 3), lambda i: (i, 0)),
            pl.BlockSpec((1, BN, 1), lambda i: (i, 0, 0)),
            pl.BlockSpec((d, h), lambda i: (0, 0)),
            pl.BlockSpec((3, d), lambda i: (0, 0)),
            pl.BlockSpec((1, h), lambda i: (0, 0)),
            pl.BlockSpec((1, h), lambda i: (0, 0)),
            pl.BlockSpec((1, 1), lambda i: (0, 0)),
        ],
        out_specs=[
            pl.BlockSpec((g, 2), lambda i: (0, 0)),
            pl.BlockSpec((3, BN), lambda i: (0, i)),
        ],
        out_shape=[
            jax.ShapeDtypeStruct((g, 2), jnp.float32),
            jax.ShapeDtypeStruct((3, npad), jnp.float32),
        ],
    )(nf, pos, ids3, w1, wp, w2r, b1r, b2r)


def _make_edge_kernel(e, n, npad, g):
    nchunks = e // CHUNK
    chunks_per = -(-nchunks // NSUB)
    ndma = CHUNK // IDXW
    nvec = CHUNK // 16
    mesh = plsc.VectorSubcoreMesh(core_axis_name="c", subcore_axis_name="s")

    @functools.partial(
        pl.kernel,
        mesh=mesh,
        compiler_params=pltpu.CompilerParams(needs_layout_passes=False,
                                             use_tc_tiling_on_sc=False),
        out_type=jax.ShapeDtypeStruct((NSUB * 9 * g,), jnp.float32),
        scratch_types=[
            pltpu.VMEM((n,), jnp.int32),
            pltpu.VMEM((1, CHUNK), jnp.int32),
            pltpu.VMEM((ndma, IDXW), jnp.int32),
            pltpu.VMEM((2 * ndma, IDXW), jnp.int32),
            pltpu.VMEM((CHUNK, 3), jnp.float32),
            pltpu.VMEM((CHUNK,), jnp.float32),
            pltpu.VMEM((CHUNK,), jnp.float32),
            pltpu.VMEM((CHUNK,), jnp.float32),
            pltpu.VMEM((9 * g,), jnp.float32),
            pltpu.SemaphoreType.DMA,
        ],
    )
    def edge_kernel(ei_hbm, ea_hbm, f_hbm, batch_hbm, zero_hbm, out_hbm,
                    batch_v, row_v, col_v, idx2_v, a_v, f0_v, f1_v, f2_v,
                    acc_v, sem):
        cid0 = lax.axis_index("s") * 2 + lax.axis_index("c")
        pltpu.sync_copy(batch_hbm, batch_v)
        pltpu.sync_copy(zero_hbm, acc_v)
        iota = lax.iota(jnp.int32, 16)

        def chunk_body(ci, carry):
            cid = ci * NSUB + cid0

            @pl.when(cid < nchunks)
            def _():
                start = cid * CHUNK
                pltpu.sync_copy(ei_hbm.at[pl.ds(0, 1), pl.ds(start, CHUNK)],
                                row_v)
                for j in range(ndma):
                    pltpu.sync_copy(
                        ei_hbm.at[pl.ds(1, 1), pl.ds(start + j * IDXW, IDXW)],
                        col_v.at[pl.ds(j, 1)])
                pltpu.sync_copy(ea_hbm.at[pl.ds(start, CHUNK)], a_v)
                for j in range(ndma):
                    for l in range(IDXW // 16):
                        c16 = col_v[j, pl.ds(l * 16, 16)]
                        idx2_v[j, pl.ds(l * 16, 16)] = c16 + npad
                        idx2_v[ndma + j, pl.ds(l * 16, 16)] = c16 + 2 * npad
                copies = []
                for j in range(ndma):
                    copies.append(pltpu.async_copy(
                        f_hbm.at[col_v.at[j]],
                        f0_v.at[pl.ds(j * IDXW, IDXW)], sem))
                    copies.append(pltpu.async_copy(
                        f_hbm.at[idx2_v.at[j]],
                        f1_v.at[pl.ds(j * IDXW, IDXW)], sem))
                    copies.append(pltpu.async_copy(
                        f_hbm.at[idx2_v.at[ndma + j]],
                        f2_v.at[pl.ds(j * IDXW, IDXW)], sem))
                for c in copies:
                    c.wait()

                def vec_body(v, carry2):
                    base = v * 16
                    lane = iota + base
                    r16 = row_v[0, pl.ds(base, 16)]
                    gid = plsc.load_gather(batch_v, [r16])
                    a = [plsc.load_gather(a_v, [lane, jnp.full((16,), k, jnp.int32)])
                         for k in range(3)]
                    f = [f0_v[pl.ds(base, 16)], f1_v[pl.ds(base, 16)],
                         f2_v[pl.ds(base, 16)]]
                    for ii in range(3):
                        for jj in range(3):
                            plsc.addupdate_scatter(
                                acc_v, [gid + (3 * ii + jj) * g], a[ii] * f[jj])
                    return carry2

                lax.fori_loop(0, nvec, vec_body, 0)
            return carry

        lax.fori_loop(0, chunks_per, chunk_body, 0)
        pltpu.sync_copy(acc_v, out_hbm.at[pl.ds(cid0 * 9 * g, 9 * g)])

    return edge_kernel


def _combine_body(p_ref, c_ref, out_ref):
    s = jnp.sum(p_ref[...], axis=0)
    c = c_ref[...]
    det = (c[:, 0] * (c[:, 4] * c[:, 8] - c[:, 5] * c[:, 7])
           - c[:, 1] * (c[:, 3] * c[:, 8] - c[:, 5] * c[:, 6])
           + c[:, 2] * (c[:, 3] * c[:, 7] - c[:, 4] * c[:, 6]))
    vol = jnp.maximum(jnp.abs(det), 1e-10)
    out_ref[...] = s / vol[None, :]


def _combine_call(partials, cell9, g):
    return pl.pallas_call(
        _combine_body,
        grid=(1,),
        in_specs=[
            pl.BlockSpec((NSUB, 9, g), lambda i: (0, 0, 0)),
            pl.BlockSpec((g, 9), lambda i: (0, 0)),
        ],
        out_specs=pl.BlockSpec((9, g), lambda i: (0, 0)),
        out_shape=jax.ShapeDtypeStruct((9, g), jnp.float32),
    )(partials, cell9)


def kernel(node_feats, pos, edge_index, edge_attr, batch, cell, W1, b1, W2, b2, Wp):
    n, d = node_feats.shape
    e = edge_index.shape[1]
    g = cell.shape[0]
    h = W1.shape[1]

    nb = -(-n // BN)
    npad = nb * BN
    ids3 = jnp.pad(batch, (0, npad - n), constant_values=-1).reshape(nb, BN, 1)
    w2r = W2.reshape(1, h)
    b1r = b1.reshape(1, h)
    b2r = b2.reshape(1, 1)

    eacc, fpad = _node_call(node_feats, pos, ids3, W1, Wp, w2r, b1r, b2r, g)

    zero = jnp.zeros((9 * g,), jnp.float32)
    edge_kernel = _make_edge_kernel(e, n, npad, g)
    partials = edge_kernel(edge_index, edge_attr, fpad.reshape(-1), batch, zero)

    stress9 = _combine_call(partials.reshape(NSUB, 9, g), cell.reshape(g, 9), g)
    return eacc[:, 0], eacc[:, 1], stress9.T.reshape(g, 3, 3)
